# Initial kernel scaffold; baseline (speedup 1.0000x reference)
import jax
import jax.numpy as jnp
from jax.experimental import pallas as pl


def kernel(current_node, encoded_input, parents):
    raise NotImplementedError("write your pallas kernel here")


# speedup vs baseline: 1.0832x; 1.0832x over previous
import functools

import jax
import jax.numpy as jnp
from jax import lax
from jax.experimental import pallas as pl
from jax.experimental.pallas import tpu as pltpu
from jax.experimental.pallas import tpu_sc as plsc

B, S, D = 4096, 200, 128
_INFO = plsc.get_sparse_core_info()
NC, NS, L = _INFO.num_cores, _INFO.num_subcores, _INFO.num_lanes
NW = NC * NS
BPW = B // NW
_MESH = plsc.VectorSubcoreMesh(core_axis_name="c", subcore_axis_name="s")


@functools.partial(
    pl.kernel,
    out_type=jax.ShapeDtypeStruct((B, D), jnp.float32),
    mesh=_MESH,
    scratch_types=[
        pltpu.VMEM((BPW,), jnp.int32),
        pltpu.VMEM((BPW, D), jnp.float32),
        pltpu.VMEM((BPW, D), jnp.float32),
        pltpu.SemaphoreType.DMA,
    ],
)
def _sc_kernel(cur_hbm, enc_hbm, par_hbm, out_hbm, idx_v, rows_v, cur_v, sem):
    wid = lax.axis_index("s") * NC + lax.axis_index("c")
    base = wid * BPW

    pltpu.sync_copy(par_hbm.at[pl.ds(base, BPW)], idx_v)
    for j in range(BPW // L):
        b_vec = base + j * L + lax.iota(jnp.int32, L)
        idx_v[pl.ds(j * L, L)] = idx_v[pl.ds(j * L, L)] + b_vec * S

    gather = pltpu.async_copy(enc_hbm.at[idx_v], rows_v, sem)
    pltpu.sync_copy(cur_hbm.at[pl.ds(base, BPW)], cur_v)
    gather.wait()

    def add_row(r, _):
        for c in range(D // L):
            sl = pl.ds(c * L, L)
            rows_v[r, sl] = rows_v[r, sl] + cur_v[r, sl]
        return 0

    lax.fori_loop(0, BPW, add_row, 0)

    pltpu.sync_copy(rows_v, out_hbm.at[pl.ds(base, BPW)])


def kernel(current_node, encoded_input, parents):
    enc_flat = encoded_input.reshape(B * S, D)
    par32 = parents.astype(jnp.int32)
    return _sc_kernel(current_node, enc_flat, par32)
